# Initial kernel scaffold; baseline (speedup 1.0000x reference)
import jax
import jax.numpy as jnp
from jax.experimental import pallas as pl


def kernel(x, tables):
    raise NotImplementedError("write your pallas kernel here")


# speedup vs baseline: 1.0361x; 1.0361x over previous
import functools

import jax
import jax.numpy as jnp
from jax import lax
from jax.experimental import pallas as pl
from jax.experimental.pallas import tpu as pltpu
from jax.experimental.pallas import tpu_sc as plsc

_NUM_FIELDS = 26
_VOCAB = 100000
_EMB = 64
_BATCH = 4096
_TOTAL = _BATCH * _NUM_FIELDS
_NW = 32
_PER_W = _TOTAL // _NW
_CHUNK = 128
_NCHUNK = _PER_W // _CHUNK
_LANES = 16
_NSLICE = _CHUNK // _LANES

_mesh = plsc.VectorSubcoreMesh(core_axis_name="c", subcore_axis_name="s")


@functools.partial(
    pl.kernel,
    mesh=_mesh,
    out_type=jax.ShapeDtypeStruct((_TOTAL, _EMB), jnp.float32),
    scratch_types=[
        pltpu.VMEM((_NCHUNK, _CHUNK), jnp.int32),
        pltpu.VMEM((_CHUNK, _EMB), jnp.float32),
        pltpu.SemaphoreType.DMA,
    ],
    compiler_params=pltpu.CompilerParams(use_tc_tiling_on_sc=False),
)
def _emb_gather(x_hbm, table_hbm, out_hbm, idx_v, rows_v, gsem):
    wid = lax.axis_index("s") * 2 + lax.axis_index("c")
    base = wid * _PER_W

    pltpu.sync_copy(x_hbm.at[wid], idx_v)

    lanes = lax.iota(jnp.int32, _LANES)

    def idx_body(c, _):
        for s in range(_NSLICE):
            sl = pl.ds(s * _LANES, _LANES)
            pos = base + c * _CHUNK + s * _LANES + lanes
            f = lax.rem(pos, _NUM_FIELDS)
            idx_v[c, sl] = idx_v[c, sl] + f * _VOCAB
        return 0

    lax.fori_loop(0, _NCHUNK, idx_body, 0)

    def gather_body(c, _):
        pltpu.async_copy(table_hbm.at[idx_v.at[c]], rows_v, gsem).wait()
        pltpu.sync_copy(rows_v, out_hbm.at[pl.ds(base + c * _CHUNK, _CHUNK)])
        return 0

    lax.fori_loop(0, _NCHUNK, gather_body, 0)


def kernel(x, tables):
    x3 = x.reshape(_NW, _NCHUNK, _CHUNK)
    table_flat = tables.reshape(_NUM_FIELDS * _VOCAB, _EMB)
    out = _emb_gather(x3, table_flat)
    return out.reshape(_BATCH, _NUM_FIELDS * _EMB)


# speedup vs baseline: 3.8558x; 3.7216x over previous
import functools

import jax
import jax.numpy as jnp
from jax import lax
from jax.experimental import pallas as pl
from jax.experimental.pallas import tpu as pltpu
from jax.experimental.pallas import tpu_sc as plsc

_NUM_FIELDS = 26
_VOCAB = 100000
_EMB = 64
_BATCH = 4096
_ROWS = _NUM_FIELDS * _EMB
_NW = 32
_PER_W = _ROWS // _NW
_LANES = 16
_NSLICE = _BATCH // _LANES

_mesh = plsc.VectorSubcoreMesh(core_axis_name="c", subcore_axis_name="s")


@functools.partial(
    pl.kernel,
    mesh=_mesh,
    out_type=jax.ShapeDtypeStruct((_ROWS, _BATCH), jnp.float32),
    scratch_types=[
        pltpu.VMEM((_VOCAB,), jnp.float32),
        pltpu.VMEM((_BATCH,), jnp.int32),
        pltpu.VMEM((_BATCH,), jnp.float32),
    ],
    compiler_params=pltpu.CompilerParams(
        use_tc_tiling_on_sc=True, needs_layout_passes=False
    ),
)
def _emb_rows(x_t, table, out_t, row_v, idx_v, val_v):
    wid = lax.axis_index("s") * 2 + lax.axis_index("c")
    r0 = wid * _PER_W

    def row_body(i, _):
        r = r0 + i
        f = r // _EMB
        pltpu.sync_copy(x_t.at[f], idx_v)
        pltpu.sync_copy(table.at[r], row_v)

        def g_body(t, _):
            sl = pl.ds(t * _LANES, _LANES)
            val_v[sl] = plsc.load_gather(row_v, [idx_v[sl]])
            return 0

        lax.fori_loop(0, _NSLICE, g_body, 0)
        pltpu.sync_copy(val_v, out_t.at[r])
        return 0

    lax.fori_loop(0, _PER_W, row_body, 0)


def kernel(x, tables):
    table2d = jnp.swapaxes(tables, 1, 2).reshape(_ROWS, _VOCAB)
    x_t = x.T
    out_t = _emb_rows(x_t, table2d)
    return out_t.T.reshape(_BATCH, _ROWS)


# speedup vs baseline: 4.6844x; 1.2149x over previous
import functools

import jax
import jax.numpy as jnp
from jax import lax
from jax.experimental import pallas as pl
from jax.experimental.pallas import tpu as pltpu
from jax.experimental.pallas import tpu_sc as plsc

_NUM_FIELDS = 26
_VOCAB = 100000
_EMB = 64
_BATCH = 4096
_ROWS = _NUM_FIELDS * _EMB
_NW = 32
_PER_W = _ROWS // _NW
_LANES = 16
_NSLICE = _BATCH // _LANES

_mesh = plsc.VectorSubcoreMesh(core_axis_name="c", subcore_axis_name="s")


@functools.partial(
    pl.kernel,
    mesh=_mesh,
    out_type=jax.ShapeDtypeStruct((_ROWS, _BATCH), jnp.float32),
    scratch_types=[
        pltpu.VMEM((_VOCAB,), jnp.float32),
        pltpu.VMEM((_BATCH,), jnp.int32),
        pltpu.VMEM((_BATCH,), jnp.float32),
        pltpu.SemaphoreType.DMA,
    ],
    compiler_params=pltpu.CompilerParams(
        use_tc_tiling_on_sc=True, needs_layout_passes=False
    ),
)
def _emb_rows(x_t, table, out_t, row_v, idx_v, val_v, wsem):
    wid = lax.axis_index("s") * 2 + lax.axis_index("c")
    r0 = wid * _PER_W
    _UNROLL = 4

    def row_body(i, prev_f):
        r = r0 + i
        f = r // _EMB

        @pl.when(f != prev_f)
        def _():
            pltpu.sync_copy(x_t.at[f], idx_v)

        pltpu.sync_copy(table.at[r], row_v)

        @pl.when(i > 0)
        def _():
            pltpu.make_async_copy(val_v, out_t.at[r - 1], wsem).wait()

        def g_body(t, _):
            for u in range(_UNROLL):
                sl = pl.ds((t * _UNROLL + u) * _LANES, _LANES)
                val_v[sl] = plsc.load_gather(row_v, [idx_v[sl]])
            return 0

        lax.fori_loop(0, _NSLICE // _UNROLL, g_body, 0)
        pltpu.async_copy(val_v, out_t.at[r], wsem)
        return f

    lax.fori_loop(0, _PER_W, row_body, -1)
    pltpu.make_async_copy(val_v, out_t.at[r0 + _PER_W - 1], wsem).wait()


def kernel(x, tables):
    table2d = jnp.swapaxes(tables, 1, 2).reshape(_ROWS, _VOCAB)
    x_t = x.T
    out_t = _emb_rows(x_t, table2d)
    return out_t.T.reshape(_BATCH, _ROWS)


# speedup vs baseline: 4.9657x; 1.0600x over previous
import functools

import jax
import jax.numpy as jnp
from jax import lax
from jax.experimental import pallas as pl
from jax.experimental.pallas import tpu as pltpu
from jax.experimental.pallas import tpu_sc as plsc

_NUM_FIELDS = 26
_VOCAB = 100000
_EMB = 64
_BATCH = 4096
_ROWS = _NUM_FIELDS * _EMB
_NW = 32
_PER_W = _ROWS // _NW
_LANES = 16
_NSLICE = _BATCH // _LANES
_H0 = 50048
_H1 = _VOCAB - _H0
_LCAP = _BATCH + _LANES

_mesh = plsc.VectorSubcoreMesh(core_axis_name="c", subcore_axis_name="s")


@functools.partial(
    pl.kernel,
    mesh=_mesh,
    out_type=jax.ShapeDtypeStruct((_ROWS, _BATCH), jnp.float32),
    scratch_types=[
        pltpu.VMEM((_H0,), jnp.float32),
        pltpu.VMEM((_H1,), jnp.float32),
        pltpu.VMEM((_BATCH,), jnp.int32),
        pltpu.VMEM((_LCAP,), jnp.int32),
        pltpu.VMEM((_LCAP,), jnp.int32),
        pltpu.VMEM((_BATCH,), jnp.float32),
        pltpu.SemaphoreType.DMA,
        pltpu.SemaphoreType.DMA,
        pltpu.SemaphoreType.DMA,
    ],
    compiler_params=pltpu.CompilerParams(
        use_tc_tiling_on_sc=True, needs_layout_passes=False
    ),
)
def _emb_rows(x_t, table, out_t, b0, b1, idx_v, l0, l1, val_v, s0, s1, wsem):
    wid = lax.axis_index("s") * 2 + lax.axis_index("c")
    r0 = wid * _PER_W
    lanes = lax.iota(jnp.int32, _LANES)

    def do_span(f, lo, hi):
        pltpu.sync_copy(x_t.at[f], idx_v)

        def pbody(t, ptrs):
            p0, p1 = ptrs
            iv = idx_v[pl.ds(t * _LANES, _LANES)]
            posf = lax.shift_left(t * _LANES + lanes, 16)
            m0 = iv < _H0
            plsc.store_compressed(l0.at[pl.ds(p0, _LANES)], iv | posf, mask=m0)
            plsc.store_compressed(
                l1.at[pl.ds(p1, _LANES)], (iv - _H0) | posf, mask=~m0
            )
            c0 = jnp.sum(jnp.where(m0, 1, 0))
            return p0 + c0, p1 + (_LANES - c0)

        n0, n1 = lax.fori_loop(0, _NSLICE, pbody, (0, 0))

        def issue0(r):
            return pltpu.async_copy(table.at[r, pl.ds(0, _H0)], b0, s0)

        def issue1(r):
            return pltpu.async_copy(table.at[r, pl.ds(_H0, _H1)], b1, s1)

        issue0(lo)
        issue1(lo)

        def gpass(buf, lst, n, size):
            def gbody(t, _):
                pk = lst[pl.ds(t * _LANES, _LANES)]
                off = jnp.minimum(pk & 0xFFFF, size - 1)
                pos = lax.shift_right_logical(pk, 16)
                tm = t * _LANES + lanes < n
                g = plsc.load_gather(buf, [off], mask=tm)
                plsc.store_scatter(val_v, [pos], g, mask=tm)
                return 0

            lax.fori_loop(0, (n + _LANES - 1) // _LANES, gbody, 0)

        def row_body(r, _):
            pltpu.make_async_copy(table.at[r, pl.ds(0, _H0)], b0, s0).wait()

            @pl.when(r > r0)
            def _():
                pltpu.make_async_copy(val_v, out_t.at[r - 1], wsem).wait()

            gpass(b0, l0, n0, _H0)

            @pl.when(r + 1 < hi)
            def _():
                issue0(r + 1)

            pltpu.make_async_copy(table.at[r, pl.ds(_H0, _H1)], b1, s1).wait()
            gpass(b1, l1, n1, _H1)

            @pl.when(r + 1 < hi)
            def _():
                issue1(r + 1)

            pltpu.async_copy(val_v, out_t.at[r], wsem)
            return 0

        lax.fori_loop(lo, hi, row_body, 0)

    f0 = r0 // _EMB
    mid = jnp.minimum(r0 + _PER_W, (f0 + 1) * _EMB)
    do_span(f0, r0, mid)

    @pl.when(mid < r0 + _PER_W)
    def _():
        do_span(f0 + 1, mid, r0 + _PER_W)

    pltpu.make_async_copy(val_v, out_t.at[r0 + _PER_W - 1], wsem).wait()


def kernel(x, tables):
    table2d = jnp.swapaxes(tables, 1, 2).reshape(_ROWS, _VOCAB)
    x_t = x.T
    out_t = _emb_rows(x_t, table2d)
    return out_t.T.reshape(_BATCH, _ROWS)
